# baseline (speedup 1.0000x reference)
import functools

import jax
import jax.numpy as jnp
from jax import lax
from jax.experimental import pallas as pl
from jax.experimental.pallas import tpu as pltpu
from jax.experimental.pallas import tpu_sc as plsc

_LAMDA = 0.9
_N = 16384
_C = 1000
_NS = 16
_L = 16
_BPW = _N // _NS
_IDXW = 128
_IDX_ROWS = _BPW // _IDXW

_mesh = plsc.VectorSubcoreMesh(core_axis_name="c", subcore_axis_name="s",
                               num_cores=1)


def _lane_shuffle(v, perm):
    return lax.gather(
        v, perm[:, None],
        dimension_numbers=lax.GatherDimensionNumbers(
            offset_dims=(), collapsed_slice_dims=(0,), start_index_map=(0,)),
        slice_sizes=(1,),
        mode=lax.GatherScatterMode.PROMISE_IN_BOUNDS)


@functools.partial(
    pl.kernel,
    out_type=(jax.ShapeDtypeStruct((_NS, _L), jnp.float32),
              jax.ShapeDtypeStruct((_L,), jnp.float32)),
    mesh=_mesh,
    scratch_types=[
        pltpu.VMEM((_BPW,), jnp.int32),
        pltpu.VMEM((_IDX_ROWS, _IDXW), jnp.int32),
        pltpu.VMEM((_IDX_ROWS, _IDXW), jnp.float32),
        pltpu.VMEM((_BPW,), jnp.float32),
        pltpu.VMEM((_BPW,), jnp.float32),
        pltpu.VMEM((_L,), jnp.float32),
        pltpu.VMEM((_NS, _L), jnp.float32),
        pltpu.SemaphoreType.DMA,
    ],
)
def _loss_kernel(table_hbm, action_hbm, delta_hbm, prop_hbm, part_hbm,
                 res_hbm, act_v, idx_v, sel_v, delta_v, prop_v, stage_v,
                 all_v, sem):
    wid = lax.axis_index("s")
    base = wid * _BPW

    pltpu.sync_copy(action_hbm.at[pl.ds(base, _BPW)], act_v)
    pltpu.sync_copy(delta_hbm.at[pl.ds(base, _BPW)], delta_v)
    pltpu.sync_copy(prop_hbm.at[pl.ds(base, _BPW)], prop_v)

    lanes = lax.iota(jnp.int32, _L)
    for t0 in range(0, _BPW, _L):
        a16 = act_v[pl.ds(t0, _L)]
        rows = (base + t0) + lanes
        idx_v[t0 // _IDXW, pl.ds(t0 % _IDXW, _L)] = rows * _C + a16

    for r in range(_IDX_ROWS):
        pltpu.async_copy(table_hbm.at[idx_v.at[r]], sel_v.at[r], sem).wait()

    acc = jnp.zeros((_L,), jnp.float32)
    for t0 in range(0, _BPW, _L):
        sel16 = sel_v[t0 // _IDXW, pl.ds(t0 % _IDXW, _L)]
        d16 = delta_v[pl.ds(t0, _L)]
        p16 = prop_v[pl.ds(t0, _L)]
        acc = acc + (-d16 - _LAMDA) * (sel16 / p16)

    stage_v[...] = acc
    pltpu.sync_copy(stage_v, part_hbm.at[wid])
    plsc.subcore_barrier()

    @pl.when(wid == 0)
    def _finalize():
        pltpu.sync_copy(part_hbm, all_v)
        tot = jnp.zeros((_L,), jnp.float32)
        for rr in range(_NS):
            tot = tot + all_v[rr, :]
        for k in (8, 4, 2, 1):
            tot = tot + _lane_shuffle(tot, lanes ^ k)
        stage_v[...] = tot * (1.0 / _N)
        pltpu.sync_copy(stage_v, res_hbm)


def kernel(output, action, delta, prop):
    flat = output.reshape(-1)
    _, res = _loss_kernel(flat, action.astype(jnp.int32), delta, prop)
    return res[0]


# speedup vs baseline: 1.7946x; 1.7946x over previous
import functools

import jax
import jax.numpy as jnp
from jax import lax
from jax.experimental import pallas as pl

_LAMDA = 0.9
_N = 16384
_C = 1000
_BR = 1024


def _loss_step(tbl_ref, act_ref, delta_ref, prop_ref, out_ref):
    pid = pl.program_id(0)

    tbl = tbl_ref[...]
    act = act_ref[...]
    cols = lax.broadcasted_iota(jnp.int32, (_BR, _C), 1)
    mask = cols == act[:, None]
    sel = jnp.sum(jnp.where(mask, tbl, 0.0), axis=1)
    w = (-delta_ref[...] - _LAMDA) / prop_ref[...]
    partial = jnp.sum(sel * w).reshape(1, 1) * (1.0 / _N)

    @pl.when(pid == 0)
    def _init():
        out_ref[...] = jnp.zeros_like(out_ref)

    out_ref[...] += partial


@jax.jit
def kernel(output, action, delta, prop):
    out = pl.pallas_call(
        _loss_step,
        grid=(_N // _BR,),
        in_specs=[
            pl.BlockSpec((_BR, _C), lambda i: (i, 0)),
            pl.BlockSpec((_BR,), lambda i: (i,)),
            pl.BlockSpec((_BR,), lambda i: (i,)),
            pl.BlockSpec((_BR,), lambda i: (i,)),
        ],
        out_specs=pl.BlockSpec((1, 1), lambda i: (0, 0)),
        out_shape=jax.ShapeDtypeStruct((1, 1), jnp.float32),
    )(output, action.astype(jnp.int32), delta, prop)
    return out[0, 0]
